# baseline (speedup 1.0000x reference)
import functools

import jax
import jax.numpy as jnp
import numpy as np
from jax import lax
from jax.experimental import pallas as pl
from jax.experimental.pallas import tpu as pltpu
from jax.experimental.pallas import tpu_sc as plsc

ROWS, COLS = 16384, 1000
COLS_PAD = 1024
NCH = COLS_PAD // 128
BLOCK_R = 1024
NBLK = ROWS // BLOCK_R
STRIPS = BLOCK_R // 8
SCOLS = NBLK * STRIPS
TOTAL = float(ROWS * COLS)
NWORK = 32
PER_W = ROWS // NWORK

_THR = tuple(float(np.float32(np.log(i / (10.0 - i)))) for i in range(1, 10))
_NEG = -1e30


def _gather_t(x_flat, flat_idx):
    mesh = plsc.VectorSubcoreMesh(core_axis_name="c", subcore_axis_name="s")

    @functools.partial(
        pl.kernel, mesh=mesh,
        out_type=jax.ShapeDtypeStruct((ROWS,), jnp.float32),
        scratch_types=[
            pltpu.VMEM((128,), jnp.int32),
            pltpu.VMEM((128,), jnp.float32),
            pltpu.SemaphoreType.DMA,
        ],
    )
    def gk(x_hbm, idx_hbm, out_hbm, idx_v, val_v, sem):
        wid = lax.axis_index("s") * 2 + lax.axis_index("c")
        base = wid * PER_W
        for j in range(PER_W // 128):
            off = base + j * 128
            pltpu.sync_copy(idx_hbm.at[pl.ds(off, 128)], idx_v)
            pltpu.async_copy(x_hbm.at[idx_v], val_v, sem).wait()
            pltpu.sync_copy(val_v, out_hbm.at[pl.ds(off, 128)])

    return gk(x_flat, flat_idx)


def _tree(vals):
    vals = list(vals)
    while len(vals) > 1:
        vals = [a + b for a, b in zip(vals[::2], vals[1::2])]
    return vals[0]


def _main_kernel(x_ref, t_ref, o_ref, d_ref, cnt_ref):
    pid = pl.program_id(0)
    lane = jax.lax.broadcasted_iota(jnp.int32, (8, 128), 1)
    tail_pad = lane >= (COLS - 896)
    zero = jnp.zeros((8, 128), jnp.float32)
    cnt_accs = [zero] * 9
    d_lane = [zero] * 10

    for s in range(STRIPS):
        xs = x_ref[s * 8:s * 8 + 8, :]
        chunks = []
        for c in range(NCH):
            ch = xs[:, c * 128:(c + 1) * 128]
            if c == NCH - 1:
                ch = jnp.where(tail_pad, _NEG, ch)
            chunks.append(ch)
        exs = [jnp.exp(ch) for ch in chunks]
        here = lane == s
        r0 = jnp.sum(_tree(exs), axis=1, keepdims=True)
        d_lane[0] = jnp.where(here, r0, d_lane[0])
        for k, thr in enumerate(_THR):
            masks = [ch >= thr for ch in chunks]
            mex = _tree([jnp.where(m, ex, 0.0)
                         for m, ex in zip(masks, exs)])
            rk = jnp.sum(mex, axis=1, keepdims=True)
            d_lane[k + 1] = jnp.where(here, rk, d_lane[k + 1])
            cnt_accs[k] = cnt_accs[k] + _tree(
                [jnp.where(m, 1.0, 0.0) for m in masks])

    for k in range(10):
        d_ref[k, :, pl.ds(pid * STRIPS, STRIPS)] = d_lane[k]

    lane16 = jax.lax.broadcasted_iota(jnp.int32, (1, 16), 1)
    cvec = jnp.zeros((1, 16), jnp.float32)
    for k in range(9):
        cvec = jnp.where(lane16 == k, jnp.sum(cnt_accs[k]), cvec)

    @pl.when(pid == 0)
    def _init():
        cnt_ref[...] = cvec

    @pl.when(pid != 0)
    def _acc():
        cnt_ref[...] += cvec

    @pl.when(pid == NBLK - 1)
    def _finalize():
        t = t_ref[...]
        nt = -t
        svec = cnt_ref[...]
        cor = jnp.zeros((1, 16), jnp.float32)
        for k, thr in enumerate(_THR):
            d = (jnp.sum(jnp.where(nt >= thr, 1.0, 0.0)) -
                 jnp.sum(jnp.where(t >= thr, 1.0, 0.0)))
            cor = jnp.where(lane16 == k, d, cor)
        sv = svec + cor
        sl = [sv[:, k:k + 1] for k in range(9)]
        cent = ([jnp.full((1, 1), TOTAL, jnp.float32) - sl[0]] +
                [sl[k - 1] - sl[k] for k in range(1, 9)] + [sl[8]])
        cbt = jnp.zeros((8, SCOLS), jnp.float32) + cent[0]
        cbx = jnp.zeros((8, SCOLS), jnp.float32) + cent[0]
        for k, thr in enumerate(_THR):
            cbt = jnp.where(nt >= thr, cent[k + 1], cbt)
            cbx = jnp.where(t >= thr, cent[k + 1], cbx)
        rec = 1.0 / cbt
        ssum = jnp.exp(t) * (1.0 - jnp.minimum(cbx * rec, 1.0))
        for b in range(10):
            eb = d_ref[b, :, :]
            if b < 9:
                eb = eb - d_ref[b + 1, :, :]
            ssum = ssum + jnp.minimum(cent[b] * rec, 1.0) * eb
        o_ref[0, 0] = jnp.sum(jnp.log(ssum) - t) / np.float32(ROWS)


def kernel(x, target):
    tgt = target.astype(jnp.int32)
    flat_idx = jnp.arange(ROWS, dtype=jnp.int32) * COLS + tgt
    t = _gather_t(x.reshape(-1), flat_idx)
    t2 = t.reshape(NBLK, STRIPS, 8).transpose(2, 0, 1).reshape(8, SCOLS)

    loss = pl.pallas_call(
        _main_kernel,
        grid=(NBLK,),
        in_specs=[
            pl.BlockSpec((BLOCK_R, COLS_PAD), lambda i: (i, 0)),
            pl.BlockSpec((8, SCOLS), lambda i: (0, 0)),
        ],
        out_specs=pl.BlockSpec((1, 1), lambda i: (0, 0),
                               memory_space=pltpu.SMEM),
        out_shape=jax.ShapeDtypeStruct((1, 1), jnp.float32),
        scratch_shapes=[
            pltpu.VMEM((16, 8, SCOLS), jnp.float32),
            pltpu.VMEM((1, 16), jnp.float32),
        ],
    )(x, t2)

    return loss[0, 0]
